# baseline (speedup 1.0000x reference)
import functools

import jax
import jax.numpy as jnp
from jax import lax
from jax.experimental import pallas as pl
from jax.experimental.pallas import tpu as pltpu
from jax.experimental.pallas import tpu_sc as plsc

_MIN_SIZE = 0.01
_MAX_SIZE = 0.999
_NW = 32
_CR = 4000
_ROWS = 1_000_000
_HALF = _ROWS // 2
_UNROLL = 5


def _normalize_chunk(bufa, bufb, sv):
    f_tiny = sv * 1e12

    def blk(i, carry):
        for u in range(_UNROLL):
            ds = pl.ds((i * _UNROLL + u) * 16, 16)
            a = bufa[ds]
            b = bufb[ds]
            t = a * a + b * b
            th = 0.5 * t
            bits = plsc.bitcast(t, jnp.int32)
            bits = 0x5F3759DF - lax.shift_right_logical(bits, 1)
            y = plsc.bitcast(bits, jnp.float32)
            y = y * (1.5 - th * (y * y))
            y = y * (1.5 - th * (y * y))
            factor = jnp.where(t >= 1e-24, sv * y, f_tiny)
            bufa[ds] = a * factor
            bufb[ds] = b * factor
        return carry

    lax.fori_loop(0, _CR // (16 * _UNROLL), blk, 0)


def _make_sc_call(rows):
    mesh = plsc.VectorSubcoreMesh(core_axis_name="c", subcore_axis_name="s")

    nchunk = rows // _CR
    maxj = -(-nchunk // _NW)
    _B = 3

    @functools.partial(
        pl.kernel,
        out_type=jax.ShapeDtypeStruct((2 * rows,), jnp.float32),
        mesh=mesh,
        scratch_types=(
            [pltpu.VMEM((_CR,), jnp.float32)] * (2 * _B)
            + [pltpu.VMEM((16,), jnp.float32)]
            + [pltpu.SemaphoreType.DMA] * (2 * _B)
        ),
        compiler_params=pltpu.CompilerParams(
            needs_layout_passes=False, use_tc_tiling_on_sc=False
        ),
    )
    def run(w_hbm, s_hbm, out_hbm, *scr):
        abuf = scr[0:_B]
        bbuf = scr[_B : 2 * _B]
        sbuf = scr[2 * _B]
        si = scr[2 * _B + 1 : 3 * _B + 1]
        so = scr[3 * _B + 1 : 4 * _B + 1]

        wid = lax.axis_index("s") * 2 + lax.axis_index("c")
        pltpu.sync_copy(s_hbm, sbuf)
        sv = jnp.clip(sbuf[...], _MIN_SIZE, _MAX_SIZE)
        base = wid * _CR

        def offs(j):
            offa = pl.multiple_of(base + j * _NW * _CR, _CR)
            offb = pl.multiple_of(rows + base + j * _NW * _CR, _CR)
            return offa, offb

        def start_in(j):
            offa, offb = offs(j)
            p = j % _B
            pltpu.async_copy(w_hbm.at[pl.ds(offa, _CR)], abuf[p], si[p])
            pltpu.async_copy(w_hbm.at[pl.ds(offb, _CR)], bbuf[p], si[p])

        def wait_in(j):
            p = j % _B
            pltpu.make_async_copy(w_hbm.at[pl.ds(0, _CR)], abuf[p], si[p]).wait()
            pltpu.make_async_copy(w_hbm.at[pl.ds(0, _CR)], bbuf[p], si[p]).wait()

        def start_out(j):
            offa, offb = offs(j)
            p = j % _B
            pltpu.async_copy(abuf[p], out_hbm.at[pl.ds(offa, _CR)], so[p])
            pltpu.async_copy(bbuf[p], out_hbm.at[pl.ds(offb, _CR)], so[p])

        def wait_out(j):
            p = j % _B
            pltpu.make_async_copy(abuf[p], out_hbm.at[pl.ds(0, _CR)], so[p]).wait()
            pltpu.make_async_copy(bbuf[p], out_hbm.at[pl.ds(0, _CR)], so[p]).wait()

        def valid(j):
            return j * _NW + wid < nchunk

        start_in(0)
        if maxj > 1:
            start_in(1)
        for j in range(maxj):
            if j + 2 < maxj:
                if j - 1 >= 0:
                    wait_out(j - 1)

                @pl.when(valid(j + 2))
                def _():
                    start_in(j + 2)

            if j < maxj - 1:
                wait_in(j)
                _normalize_chunk(abuf[j % _B], bbuf[j % _B], sv)
                start_out(j)
            else:

                @pl.when(valid(j))
                def _():
                    wait_in(j)
                    _normalize_chunk(abuf[j % _B], bbuf[j % _B], sv)
                    start_out(j)

        wait_out(maxj - 3)
        wait_out(maxj - 2)

        @pl.when(valid(maxj - 1))
        def _():
            wait_out(maxj - 1)

    return run


_sc_call_half = _make_sc_call(_HALF)


def kernel(weight, scale):
    s16 = jnp.broadcast_to(scale, (16,))
    w1 = weight[:_HALF].T.reshape(2 * _HALF)
    w2 = weight[_HALF:].T.reshape(2 * _HALF)
    o1 = _sc_call_half(w1, s16)
    o2 = _sc_call_half(w2, s16)
    y1 = o1.reshape(2, _HALF).T
    y2 = o2.reshape(2, _HALF).T
    return jnp.concatenate([y1, y2], axis=0)


# speedup vs baseline: 1.1744x; 1.1744x over previous
import functools

import jax
import jax.numpy as jnp
from jax import lax
from jax.experimental import pallas as pl
from jax.experimental.pallas import tpu as pltpu
from jax.experimental.pallas import tpu_sc as plsc

_MIN_SIZE = 0.01
_MAX_SIZE = 0.999
_NW = 32
_CR = 4000
_ROWS = 1_000_000
_SPLIT = 496_000
_UNROLL = 5


def _normalize_chunk(bufa, bufb, sv):
    f_tiny = sv * 1e12

    def blk(i, carry):
        for u in range(_UNROLL):
            ds = pl.ds((i * _UNROLL + u) * 16, 16)
            a = bufa[ds]
            b = bufb[ds]
            t = a * a + b * b
            th = 0.5 * t
            bits = plsc.bitcast(t, jnp.int32)
            bits = 0x5F3759DF - lax.shift_right_logical(bits, 1)
            y = plsc.bitcast(bits, jnp.float32)
            y = y * (1.5 - th * (y * y))
            y = y * (1.5 - th * (y * y))
            factor = jnp.where(t >= 1e-24, sv * y, f_tiny)
            bufa[ds] = a * factor
            bufb[ds] = b * factor
        return carry

    lax.fori_loop(0, _CR // (16 * _UNROLL), blk, 0)


def _make_sc_call(rows):
    mesh = plsc.VectorSubcoreMesh(core_axis_name="c", subcore_axis_name="s")

    nchunk = rows // _CR
    maxj = -(-nchunk // _NW)
    _B = 3

    @functools.partial(
        pl.kernel,
        out_type=jax.ShapeDtypeStruct((2 * rows,), jnp.float32),
        mesh=mesh,
        scratch_types=(
            [pltpu.VMEM((_CR,), jnp.float32)] * (2 * _B)
            + [pltpu.VMEM((16,), jnp.float32)]
            + [pltpu.SemaphoreType.DMA] * (2 * _B)
        ),
        compiler_params=pltpu.CompilerParams(
            needs_layout_passes=False, use_tc_tiling_on_sc=False
        ),
    )
    def run(w_hbm, s_hbm, out_hbm, *scr):
        abuf = scr[0:_B]
        bbuf = scr[_B : 2 * _B]
        sbuf = scr[2 * _B]
        si = scr[2 * _B + 1 : 3 * _B + 1]
        so = scr[3 * _B + 1 : 4 * _B + 1]

        wid = lax.axis_index("s") * 2 + lax.axis_index("c")
        pltpu.sync_copy(s_hbm, sbuf)
        sv = jnp.clip(sbuf[...], _MIN_SIZE, _MAX_SIZE)
        base = wid * _CR

        def offs(j):
            offa = pl.multiple_of(base + j * _NW * _CR, _CR)
            offb = pl.multiple_of(rows + base + j * _NW * _CR, _CR)
            return offa, offb

        def start_in(j):
            offa, offb = offs(j)
            p = j % _B
            pltpu.async_copy(w_hbm.at[pl.ds(offa, _CR)], abuf[p], si[p])
            pltpu.async_copy(w_hbm.at[pl.ds(offb, _CR)], bbuf[p], si[p])

        def wait_in(j):
            p = j % _B
            pltpu.make_async_copy(w_hbm.at[pl.ds(0, _CR)], abuf[p], si[p]).wait()
            pltpu.make_async_copy(w_hbm.at[pl.ds(0, _CR)], bbuf[p], si[p]).wait()

        def start_out(j):
            offa, offb = offs(j)
            p = j % _B
            pltpu.async_copy(abuf[p], out_hbm.at[pl.ds(offa, _CR)], so[p])
            pltpu.async_copy(bbuf[p], out_hbm.at[pl.ds(offb, _CR)], so[p])

        def wait_out(j):
            p = j % _B
            pltpu.make_async_copy(abuf[p], out_hbm.at[pl.ds(0, _CR)], so[p]).wait()
            pltpu.make_async_copy(bbuf[p], out_hbm.at[pl.ds(0, _CR)], so[p]).wait()

        def valid(j):
            return j * _NW + wid < nchunk

        start_in(0)
        if maxj > 1:
            start_in(1)
        for j in range(maxj):
            if j + 2 < maxj:
                if j - 1 >= 0:
                    wait_out(j - 1)

                @pl.when(valid(j + 2))
                def _():
                    start_in(j + 2)

            if j < maxj - 1:
                wait_in(j)
                _normalize_chunk(abuf[j % _B], bbuf[j % _B], sv)
                start_out(j)
            else:

                @pl.when(valid(j))
                def _():
                    wait_in(j)
                    _normalize_chunk(abuf[j % _B], bbuf[j % _B], sv)
                    start_out(j)

        wait_out(maxj - 3)
        wait_out(maxj - 2)

        @pl.when(valid(maxj - 1))
        def _():
            wait_out(maxj - 1)

    return run


_R1 = _SPLIT
_R2 = _ROWS - _SPLIT
_sc_call_1 = _make_sc_call(_R1)
_sc_call_2 = _make_sc_call(_R2)


def kernel(weight, scale):
    s16 = jnp.broadcast_to(scale, (16,))
    w1 = weight[:_R1].T.reshape(2 * _R1)
    w2 = weight[_R1:].T.reshape(2 * _R2)
    o1 = _sc_call_1(w1, s16)
    o2 = _sc_call_2(w2, s16)
    y1 = o1.reshape(2, _R1).T
    y2 = o2.reshape(2, _R2).T
    return jnp.concatenate([y1, y2], axis=0)


# speedup vs baseline: 1.7708x; 1.5078x over previous
import functools

import jax
import jax.numpy as jnp
from jax import lax
from jax.experimental import pallas as pl
from jax.experimental.pallas import tpu as pltpu
from jax.experimental.pallas import tpu_sc as plsc

_MIN_SIZE = 0.01
_MAX_SIZE = 0.999
_NW = 32
_ROWS = 1_000_000
_BLK = 256
_NFULL = 7812
_MAIN_ROWS = _NFULL * 128
_MAIN_WORDS = _NFULL * _BLK
_OUT_WORDS = (_NFULL + 1) * _BLK
_CB = 36
_CW = _CB * _BLK
_NCHUNK = _NFULL // _CB
_MAXJ = -(-_NCHUNK // _NW)


def _pair_step(src, dst, off_a, off_b, sv, f_tiny):
    a = src[pl.ds(off_a, 16)]
    b = src[pl.ds(off_b, 16)]
    t = a * a + b * b
    th = 0.5 * t
    bits = plsc.bitcast(t, jnp.int32)
    bits = 0x5F3759DF - lax.shift_right_logical(bits, 1)
    y = plsc.bitcast(bits, jnp.float32)
    y = y * (1.5 - th * (y * y))
    y = y * (1.5 - th * (y * y))
    factor = jnp.where(t >= 1e-24, sv * y, f_tiny)
    dst[pl.ds(off_a, 16)] = a * factor
    dst[pl.ds(off_b, 16)] = b * factor


def _normalize_chunk(buf, sv):
    f_tiny = sv * 1e12

    def blk(k, carry):
        for m in range(8):
            a = buf[k, 0, pl.ds(16 * m, 16)]
            b = buf[k, 1, pl.ds(16 * m, 16)]
            t = a * a + b * b
            th = 0.5 * t
            bits = plsc.bitcast(t, jnp.int32)
            bits = 0x5F3759DF - lax.shift_right_logical(bits, 1)
            y = plsc.bitcast(bits, jnp.float32)
            y = y * (1.5 - th * (y * y))
            y = y * (1.5 - th * (y * y))
            factor = jnp.where(t >= 1e-24, sv * y, f_tiny)
            buf[k, 0, pl.ds(16 * m, 16)] = a * factor
            buf[k, 1, pl.ds(16 * m, 16)] = b * factor
        return carry

    lax.fori_loop(0, _CB, blk, 0)


def _make_sc_call():
    mesh = plsc.VectorSubcoreMesh(core_axis_name="c", subcore_axis_name="s")

    _B = 3

    @functools.partial(
        pl.kernel,
        out_type=jax.ShapeDtypeStruct((_NFULL + 1, 2, 128), jnp.float32),
        mesh=mesh,
        scratch_types=(
            [pltpu.VMEM((_CB, 2, 128), jnp.float32)] * _B
            + [pltpu.VMEM((16,), jnp.float32)]
            + [pltpu.VMEM((128,), jnp.float32)] * 2
            + [pltpu.SemaphoreType.DMA] * (2 * _B)
        ),
        compiler_params=pltpu.CompilerParams(
            needs_layout_passes=False, use_tc_tiling_on_sc=False
        ),
    )
    def run(w_hbm, tail_hbm, s_hbm, out_hbm, *scr):
        buf = scr[0:_B]
        sbuf = scr[_B]
        tin = scr[_B + 1]
        tout = scr[_B + 2]
        si = scr[_B + 3 : 2 * _B + 3]
        so = scr[2 * _B + 3 : 3 * _B + 3]

        wid = lax.axis_index("s") * 2 + lax.axis_index("c")
        pltpu.sync_copy(s_hbm, sbuf)
        sv = jnp.clip(sbuf[...], _MIN_SIZE, _MAX_SIZE)
        base = wid * _CB

        def off(j):
            return pl.multiple_of(base + j * _NW * _CB, _CB)

        def start_in(j):
            p = j % _B
            pltpu.async_copy(w_hbm.at[pl.ds(off(j), _CB)], buf[p], si[p])

        def wait_in(j):
            p = j % _B
            pltpu.make_async_copy(w_hbm.at[pl.ds(0, _CB)], buf[p], si[p]).wait()

        def start_out(j):
            p = j % _B
            pltpu.async_copy(buf[p], out_hbm.at[pl.ds(off(j), _CB)], so[p])

        def wait_out(j):
            p = j % _B
            pltpu.make_async_copy(buf[p], out_hbm.at[pl.ds(0, _CB)], so[p]).wait()

        def valid(j):
            return j * _NW + wid < _NCHUNK

        start_in(0)
        if _MAXJ > 1:
            start_in(1)
        for j in range(_MAXJ):
            if j + 2 < _MAXJ:
                if j - 1 >= 0:
                    wait_out(j - 1)

                @pl.when(valid(j + 2))
                def _():
                    start_in(j + 2)

            if j < _MAXJ - 1:
                wait_in(j)
                _normalize_chunk(buf[j % _B], sv)
                start_out(j)
            else:

                @pl.when(valid(j))
                def _():
                    wait_in(j)
                    _normalize_chunk(buf[j % _B], sv)
                    start_out(j)

        @pl.when(wid == _NW - 1)
        def _():
            pltpu.sync_copy(tail_hbm, tin)
            f_tiny = sv * 1e12
            for m in range(4):
                _pair_step(tin, tout, 16 * m, 64 + 16 * m, sv, f_tiny)
            pltpu.sync_copy(tout.at[pl.ds(0, 64)],
                            out_hbm.at[_NFULL, 0, pl.ds(0, 64)])
            pltpu.sync_copy(tout.at[pl.ds(64, 64)],
                            out_hbm.at[_NFULL, 1, pl.ds(0, 64)])

        wait_out(_MAXJ - 3)
        wait_out(_MAXJ - 2)

        @pl.when(valid(_MAXJ - 1))
        def _():
            wait_out(_MAXJ - 1)

    return run


_sc_call = _make_sc_call()


def kernel(weight, scale):
    s16 = jnp.broadcast_to(scale, (16,))
    in3 = weight[:_MAIN_ROWS].reshape(_NFULL, 128, 2).swapaxes(1, 2)
    tail_in = weight[_MAIN_ROWS:].T.reshape(128)
    o = _sc_call(in3, tail_in, s16)
    z = o.swapaxes(1, 2).reshape((_NFULL + 1) * 128, 2)
    return z[:_ROWS]


# speedup vs baseline: 1.8897x; 1.0672x over previous
import functools

import jax
import jax.numpy as jnp
from jax import lax
from jax.experimental import pallas as pl
from jax.experimental.pallas import tpu as pltpu
from jax.experimental.pallas import tpu_sc as plsc

_MIN_SIZE = 0.01
_MAX_SIZE = 0.999
_NW = 32
_ROWS = 1_000_000
_NFULL = 7812
_MAIN_ROWS = _NFULL * 128
_OUTR = 2 * (_NFULL + 4)
_CB = 36
_NCHUNK = _NFULL // _CB
_MAXJ = -(-_NCHUNK // _NW)


def _pair_step(src, dst, off_a, off_b, sv, f_tiny):
    a = src[pl.ds(off_a, 16)]
    b = src[pl.ds(off_b, 16)]
    t = a * a + b * b
    th = 0.5 * t
    bits = plsc.bitcast(t, jnp.int32)
    bits = 0x5F3759DF - lax.shift_right_logical(bits, 1)
    y = plsc.bitcast(bits, jnp.float32)
    y = y * (1.5 - th * (y * y))
    y = y * (1.5 - th * (y * y))
    factor = jnp.where(t >= 1e-24, sv * y, f_tiny)
    dst[pl.ds(off_a, 16)] = a * factor
    dst[pl.ds(off_b, 16)] = b * factor


def _normalize_chunk(bin_, bout, sv):
    f_tiny = sv * 1e12

    def blk(k, carry):
        for m in range(8):
            a = bin_[k, 0, pl.ds(16 * m, 16)]
            b = bin_[k, 1, pl.ds(16 * m, 16)]
            t = a * a + b * b
            th = 0.5 * t
            bits = plsc.bitcast(t, jnp.int32)
            bits = 0x5F3759DF - lax.shift_right_logical(bits, 1)
            y = plsc.bitcast(bits, jnp.float32)
            y = y * (1.5 - th * (y * y))
            y = y * (1.5 - th * (y * y))
            factor = jnp.where(t >= 1e-24, sv * y, f_tiny)
            bout[2 * k, pl.ds(16 * m, 16)] = a * factor
            bout[2 * k + 1, pl.ds(16 * m, 16)] = b * factor
        return carry

    lax.fori_loop(0, _CB, blk, 0)


def _make_sc_call():
    mesh = plsc.VectorSubcoreMesh(core_axis_name="c", subcore_axis_name="s")

    _B = 3

    @functools.partial(
        pl.kernel,
        out_type=jax.ShapeDtypeStruct((_OUTR, 128), jnp.float32),
        mesh=mesh,
        scratch_types=(
            [pltpu.VMEM((_CB, 2, 128), jnp.float32)] * _B
            + [pltpu.VMEM((2 * _CB, 128), jnp.float32)] * _B
            + [pltpu.VMEM((16,), jnp.float32)]
            + [pltpu.VMEM((128,), jnp.float32)] * 2
            + [pltpu.SemaphoreType.DMA] * (2 * _B)
        ),
        compiler_params=pltpu.CompilerParams(
            needs_layout_passes=False, use_tc_tiling_on_sc=False
        ),
    )
    def run(w_hbm, tail_hbm, s_hbm, out_hbm, *scr):
        bin_ = scr[0:_B]
        bout = scr[_B : 2 * _B]
        sbuf = scr[2 * _B]
        tin = scr[2 * _B + 1]
        tout = scr[2 * _B + 2]
        si = scr[2 * _B + 3 : 3 * _B + 3]
        so = scr[3 * _B + 3 : 4 * _B + 3]

        wid = lax.axis_index("s") * 2 + lax.axis_index("c")
        pltpu.sync_copy(s_hbm, sbuf)
        sv = jnp.clip(sbuf[...], _MIN_SIZE, _MAX_SIZE)

        def cid(j):
            return j * _NW + wid

        def start_in(j):
            p = j % _B
            off = pl.multiple_of(cid(j) * _CB, _CB)
            pltpu.async_copy(w_hbm.at[pl.ds(off, _CB)], bin_[p], si[p])

        def wait_in(j):
            p = j % _B
            pltpu.make_async_copy(w_hbm.at[pl.ds(0, _CB)], bin_[p], si[p]).wait()

        def start_out(j):
            p = j % _B
            off = pl.multiple_of(cid(j) * 2 * _CB, 2 * _CB)
            pltpu.async_copy(bout[p], out_hbm.at[pl.ds(off, 2 * _CB)], so[p])

        def wait_out(j):
            p = j % _B
            pltpu.make_async_copy(
                bout[p], out_hbm.at[pl.ds(0, 2 * _CB)], so[p]
            ).wait()

        def valid(j):
            return cid(j) < _NCHUNK

        start_in(0)
        if _MAXJ > 1:
            start_in(1)
        for j in range(_MAXJ):
            if j + 2 < _MAXJ:
                if j - 1 >= 0:
                    wait_out(j - 1)

                @pl.when(valid(j + 2))
                def _():
                    start_in(j + 2)

            if j < _MAXJ - 1:
                wait_in(j)
                _normalize_chunk(bin_[j % _B], bout[j % _B], sv)
                start_out(j)
            else:

                @pl.when(valid(j))
                def _():
                    wait_in(j)
                    _normalize_chunk(bin_[j % _B], bout[j % _B], sv)
                    start_out(j)

        @pl.when(wid == _NW - 1)
        def _():
            pltpu.sync_copy(tail_hbm, tin)
            f_tiny = sv * 1e12
            for m in range(4):
                _pair_step(tin, tout, 16 * m, 64 + 16 * m, sv, f_tiny)
            pltpu.sync_copy(tout.at[pl.ds(0, 64)],
                            out_hbm.at[2 * _NFULL, pl.ds(0, 64)])
            pltpu.sync_copy(tout.at[pl.ds(64, 64)],
                            out_hbm.at[2 * _NFULL + 1, pl.ds(0, 64)])

        wait_out(_MAXJ - 3)
        wait_out(_MAXJ - 2)

        @pl.when(valid(_MAXJ - 1))
        def _():
            wait_out(_MAXJ - 1)

    return run


_sc_call = _make_sc_call()


def kernel(weight, scale):
    s16 = jnp.broadcast_to(scale, (16,))
    in3 = weight[:_MAIN_ROWS].reshape(_NFULL, 128, 2).swapaxes(1, 2)
    tail_in = weight[_MAIN_ROWS:].T.reshape(128)
    o = _sc_call(in3, tail_in, s16)
    z = (
        o.reshape(_NFULL + 4, 2, 128)
        .swapaxes(1, 2)
        .reshape((_NFULL + 4) * 128, 2)
    )
    return z[:_ROWS]
